# baseline (speedup 1.0000x reference)
import functools

import jax
import jax.numpy as jnp
from jax import lax
from jax.experimental import pallas as pl
from jax.experimental.pallas import tpu as pltpu
from jax.experimental.pallas import tpu_sc as plsc

KK = 1024
DD = 64
BETA = 0.25
FBIAS = 0x3F800000

NC = 2
NS = 16
NW = NC * NS
LANES = 16
HALF = 512


def _vq_body(ze_ref, e_ref, biota_ref, idx_ref, sse_ref):
    zb = ze_ref[0]
    e = e_ref[...]
    biota = biota_ref[...]
    z2 = jnp.sum(zb * zb, axis=0, keepdims=True)
    e2 = jnp.sum(e * e, axis=1, keepdims=True)
    mm2 = lax.dot_general(e + e, zb, (((1,), (0,)), ((), ())),
                          preferred_element_type=jnp.float32)
    d = (z2 + e2) - mm2
    m = jnp.min(d, axis=0, keepdims=True)
    idxf = jnp.min(jnp.where(d == m, biota, 2.0), axis=0, keepdims=True)
    idx_ref[0] = lax.bitcast_convert_type(idxf, jnp.int32) - FBIAS

    @pl.when(pl.program_id(0) == 0)
    def _():
        sse_ref[0, 0] = 0.0

    sse_ref[0, 0] += jnp.sum(m)


def _vq(ze3, e):
    b = ze3.shape[0]
    hw = ze3.shape[2]
    biota = lax.bitcast_convert_type(
        jnp.broadcast_to(
            (jnp.arange(KK, dtype=jnp.int32) | FBIAS)[:, None], (KK, hw)
        ),
        jnp.float32,
    )
    return pl.pallas_call(
        _vq_body,
        grid=(b,),
        in_specs=[
            pl.BlockSpec((1, DD, hw), lambda i: (i, 0, 0)),
            pl.BlockSpec((KK, DD), lambda i: (0, 0)),
            pl.BlockSpec((KK, hw), lambda i: (0, 0)),
        ],
        out_specs=[
            pl.BlockSpec((1, 1, hw), lambda i: (i, 0, 0)),
            pl.BlockSpec((1, 1), lambda i: (0, 0), memory_space=pltpu.SMEM),
        ],
        out_shape=[
            jax.ShapeDtypeStruct((b, 1, hw), jnp.int32),
            jax.ShapeDtypeStruct((1, 1), jnp.float32),
        ],
    )(ze3, e, biota)


def _gather_body(etf_hbm, idx_hbm, out_hbm, tab_v, idx_v, slab_v, sem):
    wid = lax.axis_index("s") * NC + lax.axis_index("c")
    pltpu.sync_copy(etf_hbm, tab_v)
    pltpu.sync_copy(idx_hbm.at[wid], idx_v)
    for half in range(2):
        def group(g, carry):
            idxg = idx_v[pl.ds(half * HALF + g * LANES, LANES)]
            for c in range(DD):
                slab_v[c, pl.ds(g * LANES, LANES)] = plsc.load_gather(
                    tab_v, [idxg + jnp.int32(c * KK)])
            return carry

        lax.fori_loop(0, HALF // LANES, group, 0)
        pltpu.sync_copy(slab_v, out_hbm.at[wid, :, pl.ds(half * HALF, HALF)])


def _sc_gather_t(et_flat, idx2d, b, hw):
    mesh = plsc.VectorSubcoreMesh(core_axis_name="c", subcore_axis_name="s")
    fn = pl.kernel(
        _gather_body,
        out_type=jax.ShapeDtypeStruct((b, DD, hw), jnp.float32),
        mesh=mesh,
        scratch_types=[
            pltpu.VMEM((DD * KK,), jnp.float32),
            pltpu.VMEM((hw,), jnp.int32),
            pltpu.VMEM((DD, HALF), jnp.float32),
            pltpu.SemaphoreType.DMA,
        ],
        compiler_params=pltpu.CompilerParams(use_tc_tiling_on_sc=False,
                                             needs_layout_passes=False),
    )
    return fn(et_flat, idx2d)


def kernel(z_e, codebook):
    b, c, h, w = z_e.shape
    hw = h * w
    ze3 = z_e.reshape(b, c, hw)
    idx3, sse = _vq(ze3, codebook)
    et_flat = codebook.T.reshape(c * KK)
    zq3 = _sc_gather_t(et_flat, idx3.reshape(b, hw), b, hw)
    commit = BETA * (sse[0, 0] / jnp.float32(b * c * hw))
    z_q_out = zq3.reshape(b, c, h, w)
    indices_out = idx3.reshape(b, h, w)
    codebook_loss = jnp.zeros(())
    return (z_q_out, codebook_loss, commit, commit, indices_out)


# speedup vs baseline: 1.1742x; 1.1742x over previous
import functools

import jax
import jax.numpy as jnp
from jax import lax
from jax.experimental import pallas as pl
from jax.experimental.pallas import tpu as pltpu

KK = 1024
DD = 64
BETA = 0.25
FBIAS = 0x3F800000


def _vq_body(ze_ref, e_ref, et_ref, biota_ref, zq_ref, idx_ref, sse_ref):
    zb = ze_ref[0]
    e = e_ref[...]
    et = et_ref[...]
    biota = biota_ref[...]
    z2 = jnp.sum(zb * zb, axis=0, keepdims=True)
    e2 = jnp.sum(e * e, axis=1, keepdims=True)
    mm2 = lax.dot_general(e + e, zb, (((1,), (0,)), ((), ())),
                          preferred_element_type=jnp.float32)
    d = (z2 + e2) - mm2
    m = jnp.min(d, axis=0, keepdims=True)
    idxf = jnp.min(jnp.where(d == m, biota, 2.0), axis=0, keepdims=True)
    onehot = jnp.where(biota == idxf, 1.0, 0.0)
    zq = lax.dot_general(et, onehot, (((1,), (0,)), ((), ())),
                         preferred_element_type=jnp.float32)
    zq_ref[0] = zb + (zq - zb)
    idx_ref[0] = lax.bitcast_convert_type(idxf, jnp.int32) - FBIAS

    @pl.when((pl.program_id(0) == 0) & (pl.program_id(1) == 0))
    def _():
        sse_ref[0, 0] = 0.0

    sse_ref[0, 0] += jnp.sum(m)


def _vq(ze3, e, bhw):
    b = ze3.shape[0]
    hw = ze3.shape[2]
    nh = hw // bhw
    biota = lax.bitcast_convert_type(
        jnp.broadcast_to(
            (jnp.arange(KK, dtype=jnp.int32) | FBIAS)[:, None], (KK, bhw)
        ),
        jnp.float32,
    )
    return pl.pallas_call(
        _vq_body,
        grid=(b, nh),
        in_specs=[
            pl.BlockSpec((1, DD, bhw), lambda i, j: (i, 0, j)),
            pl.BlockSpec((KK, DD), lambda i, j: (0, 0)),
            pl.BlockSpec((DD, KK), lambda i, j: (0, 0)),
            pl.BlockSpec((KK, bhw), lambda i, j: (0, 0)),
        ],
        out_specs=[
            pl.BlockSpec((1, DD, bhw), lambda i, j: (i, 0, j)),
            pl.BlockSpec((1, 1, bhw), lambda i, j: (i, 0, j)),
            pl.BlockSpec((1, 1), lambda i, j: (0, 0), memory_space=pltpu.SMEM),
        ],
        out_shape=[
            jax.ShapeDtypeStruct((b, DD, hw), jnp.float32),
            jax.ShapeDtypeStruct((b, 1, hw), jnp.int32),
            jax.ShapeDtypeStruct((1, 1), jnp.float32),
        ],
    )(ze3, e, e.T, biota)


def kernel(z_e, codebook):
    b, c, h, w = z_e.shape
    hw = h * w
    ze3 = z_e.reshape(b, c, hw)
    zq3, idx3, sse = _vq(ze3, codebook, 512)
    commit = BETA * (sse[0, 0] / jnp.float32(b * c * hw))
    z_q_out = zq3.reshape(b, c, h, w)
    indices_out = idx3.reshape(b, h, w)
    codebook_loss = jnp.zeros(())
    return (z_q_out, codebook_loss, commit, commit, indices_out)


# speedup vs baseline: 1.3529x; 1.1522x over previous
import functools

import jax
import jax.numpy as jnp
from jax import lax
from jax.experimental import pallas as pl
from jax.experimental.pallas import tpu as pltpu

KK = 1024
DD = 64
BETA = 0.25
FBIAS = 0x3F800000


def _vq_body(ze_ref, e2x_ref, et_ref, e2_ref, biota_ref, zq_ref, idx_ref,
             sse_ref):
    zb = ze_ref[0]
    e2x = e2x_ref[...]
    et = et_ref[...]
    e2 = e2_ref[...]
    biota = biota_ref[...]
    z2 = jnp.sum(zb * zb, axis=0, keepdims=True)
    mm2 = lax.dot_general(e2x, zb, (((1,), (0,)), ((), ())),
                          preferred_element_type=jnp.float32)
    d = (z2 + e2) - mm2
    m = jnp.min(d, axis=0, keepdims=True)
    idxf = jnp.min(jnp.where(d == m, biota, 2.0), axis=0, keepdims=True)
    onehot = jnp.where(biota == idxf, 1.0, 0.0)
    zq = lax.dot_general(et, onehot, (((1,), (0,)), ((), ())),
                         preferred_element_type=jnp.float32)
    zq_ref[0] = zb + (zq - zb)
    idx_ref[0] = lax.bitcast_convert_type(idxf, jnp.int32) - FBIAS

    @pl.when((pl.program_id(0) == 0) & (pl.program_id(1) == 0))
    def _():
        sse_ref[0, 0] = 0.0

    sse_ref[0, 0] += jnp.sum(m)


def _vq(ze3, e, bhw):
    b = ze3.shape[0]
    hw = ze3.shape[2]
    nh = hw // bhw
    biota = lax.bitcast_convert_type(
        jnp.broadcast_to(
            (jnp.arange(KK, dtype=jnp.int32) | FBIAS)[:, None], (KK, bhw)
        ),
        jnp.float32,
    )
    return pl.pallas_call(
        _vq_body,
        grid=(b, nh),
        in_specs=[
            pl.BlockSpec((1, DD, bhw), lambda i, j: (i, 0, j)),
            pl.BlockSpec((KK, DD), lambda i, j: (0, 0)),
            pl.BlockSpec((DD, KK), lambda i, j: (0, 0)),
            pl.BlockSpec((KK, 1), lambda i, j: (0, 0)),
            pl.BlockSpec((KK, bhw), lambda i, j: (0, 0)),
        ],
        out_specs=[
            pl.BlockSpec((1, DD, bhw), lambda i, j: (i, 0, j)),
            pl.BlockSpec((1, 1, bhw), lambda i, j: (i, 0, j)),
            pl.BlockSpec((1, 1), lambda i, j: (0, 0), memory_space=pltpu.SMEM),
        ],
        out_shape=[
            jax.ShapeDtypeStruct((b, DD, hw), jnp.float32),
            jax.ShapeDtypeStruct((b, 1, hw), jnp.int32),
            jax.ShapeDtypeStruct((1, 1), jnp.float32),
        ],
    )(ze3, e + e, e.T, jnp.sum(e * e, axis=1, keepdims=True), biota)


def kernel(z_e, codebook):
    b, c, h, w = z_e.shape
    hw = h * w
    ze3 = z_e.reshape(b, c, hw)
    zq3, idx3, sse = _vq(ze3, codebook, 1024)
    commit = BETA * (sse[0, 0] / jnp.float32(b * c * hw))
    z_q_out = zq3.reshape(b, c, h, w)
    indices_out = idx3.reshape(b, h, w)
    codebook_loss = jnp.zeros(())
    return (z_q_out, codebook_loss, commit, commit, indices_out)
